# baseline (speedup 1.0000x reference)
import functools

import jax
import jax.numpy as jnp
from jax import lax
from jax.experimental import pallas as pl
from jax.experimental.pallas import tpu as pltpu
from jax.experimental.pallas import tpu_sc as plsc

H = 128
NX = 10000
E = 320000
ROW_BLOCK = 2000

NC = 2
NS = 16
NW = NC * NS
L = 16

ROWS_PER_W = 320
N_PAD = NW * ROWS_PER_W
TRASH = ROWS_PER_W
ACC_ROWS = ROWS_PER_W + 8
KBUF = 256
UNROLL = 2
FLUSH_AT = KBUF - UNROLL * L
CHUNK = 3200
N_CHUNKS = E // CHUNK

SUM_BLK_ROWS = 8
SUM_BLK = SUM_BLK_ROWS * 128
E_IDX_ROWS = -(-E // 128)
N_BLOCKS = -(-E_IDX_ROWS // SUM_BLK_ROWS)
E_IDX_ROWS_PAD = N_BLOCKS * SUM_BLK_ROWS
MAX_BLOCKS_PER_W = -(-N_BLOCKS // NW)
NY_ACC = 10016
STRIPE = 624
ZROWS = 78


def _mlp_block_kernel(a_ref, b_ref, w1t_ref, b1_ref, w2t_ref, b2_ref, o_ref,
                      *, two_b):
    a = a_ref[...]
    b = b_ref[0][...] + b_ref[1][...] if two_b else b_ref[0][...]
    h = jnp.dot(a, w1t_ref[:H, :], preferred_element_type=jnp.float32)
    h += jnp.dot(b, w1t_ref[H:, :], preferred_element_type=jnp.float32)
    h = jnp.maximum(h + b1_ref[...], 0.0)
    o = jnp.dot(h, w2t_ref[...], preferred_element_type=jnp.float32)
    o_ref[...] = jnp.maximum(o + b2_ref[...], 0.0).astype(o_ref.dtype)


def _mlp_pallas(a, bs, W1, b1, W2, b2, out_dtype=jnp.float32):
    n = a.shape[0]
    grid = n // ROW_BLOCK
    two_b = len(bs) == 2
    bspec = pl.BlockSpec((ROW_BLOCK, H), lambda i: (i, 0))
    return pl.pallas_call(
        functools.partial(_mlp_block_kernel, two_b=two_b),
        grid=(grid,),
        in_specs=[
            bspec,
            [bspec] * len(bs),
            pl.BlockSpec((2 * H, 2 * H), lambda i: (0, 0)),
            pl.BlockSpec((1, 2 * H), lambda i: (0, 0)),
            pl.BlockSpec((2 * H, H), lambda i: (0, 0)),
            pl.BlockSpec((1, H), lambda i: (0, 0)),
        ],
        out_specs=bspec,
        out_shape=jax.ShapeDtypeStruct((n, H), out_dtype),
    )(a, list(bs), W1.T, b1[None, :], W2.T, b2[None, :])


KB = 256
MAXBLK = -(-E // KB) + 1


def _compact_body(dst_hbm, src_hbm, dlist, slist, counts,
                  dstbuf, srcbuf, dbuf, sbuf, cbuf):
    wid = lax.axis_index("s") * NC + lax.axis_index("c")
    lo = wid * ROWS_PER_W
    trash_vec = jnp.full((L,), TRASH, dtype=jnp.int32)
    zero_vec = jnp.zeros((L,), dtype=jnp.int32)

    def init_bufs(i, _):
        dstbuf[pl.ds(i * L, L)] = trash_vec
        srcbuf[pl.ds(i * L, L)] = zero_vec
        return 0
    lax.fori_loop(0, KB // L, init_bufs, 0)

    def flush(nblk):
        base = (wid * MAXBLK + nblk) * KB
        pltpu.sync_copy(dstbuf, dlist.at[pl.ds(base, KB)])
        pltpu.sync_copy(srcbuf, slist.at[pl.ds(base, KB)])
        lax.fori_loop(0, KB // L, init_bufs, 0)
        return nblk + 1

    def chunk_body(c, carry):
        pltpu.sync_copy(dst_hbm.at[pl.ds(c * CHUNK, CHUNK)], dbuf)
        pltpu.sync_copy(src_hbm.at[pl.ds(c * CHUNK, CHUNK)], sbuf)

        def step(t, carry):
            off, nblk = carry
            nblk = lax.cond(off >= FLUSH_AT,
                            lambda: flush(nblk), lambda: nblk)
            off = lax.cond(off >= FLUSH_AT, lambda: jnp.int32(0), lambda: off)
            for u in range(UNROLL):
                base = t * (UNROLL * L) + u * L
                dv = dbuf[pl.ds(base, L)]
                sv = sbuf[pl.ds(base, L)]
                m = (dv >= lo) & (dv < lo + ROWS_PER_W)
                cnt = plsc.all_reduce_population_count(m)[0]
                plsc.store_compressed(dstbuf.at[pl.ds(off, L)], dv - lo,
                                      mask=m)
                plsc.store_compressed(srcbuf.at[pl.ds(off, L)], sv, mask=m)
                off = off + cnt
            return (off, nblk)

        return lax.fori_loop(0, CHUNK // (UNROLL * L), step, carry)

    off, nblk = lax.fori_loop(0, N_CHUNKS, chunk_body,
                              (jnp.int32(0), jnp.int32(0)))
    nblk = flush(nblk)
    cbuf[pl.ds(0, L)] = jnp.full((L,), 0, jnp.int32) + nblk
    pltpu.sync_copy(cbuf, counts.at[pl.ds(wid * L, L)])


def _rmw_min_body(table_ref, dlist, slist, counts, out_ref,
                  acc, rows, dstbuf, idx2d, cbuf, sem):
    wid = lax.axis_index("s") * NC + lax.axis_index("c")
    lo = wid * ROWS_PER_W
    inf_vec = jnp.full((L,), jnp.inf, dtype=jnp.float32)

    def init_acc(r, _):
        for j in range(H // L):
            acc[r, pl.ds(j * L, L)] = inf_vec
        return 0
    lax.fori_loop(0, ACC_ROWS, init_acc, 0)

    pltpu.sync_copy(counts.at[pl.ds(wid * L, L)], cbuf)
    nblk = cbuf[pl.ds(0, L)][0]

    def block_body(b, _):
        @pl.when(b < nblk)
        def _():
            base = (wid * MAXBLK + b) * KB
            pltpu.sync_copy(dlist.at[pl.ds(base, KB)], dstbuf)
            for g in range(KB // 128):
                pltpu.sync_copy(slist.at[pl.ds(base + g * 128, 128)],
                                idx2d.at[g])
            descs = []
            for g in range(KB // 128):
                descs.append(pltpu.async_copy(
                    table_ref.at[idx2d.at[g]],
                    rows.at[pl.ds(g * 128, 128)], sem))
            for d in descs:
                d.wait()

            def rmw_group(g, _):
                dvec = dstbuf[pl.ds(g * L, L)]
                for i in range(L):
                    d = dvec[i]
                    k = g * L + i
                    for j in range(H // L):
                        sl = pl.ds(j * L, L)
                        acc[d, sl] = jnp.minimum(acc[d, sl], rows[k, sl])
                return 0
            lax.fori_loop(0, KB // L, rmw_group, 0)
        return 0

    lax.fori_loop(0, MAXBLK, block_body, 0)

    inf = jnp.float32(jnp.inf)
    def finalize(r, _):
        for j in range(H // L):
            sl = pl.ds(j * L, L)
            v = acc[r, sl]
            acc[r, sl] = jnp.where(v < inf, v, 0.0)
        return 0
    lax.fori_loop(0, ROWS_PER_W, finalize, 0)

    pltpu.sync_copy(acc.at[pl.ds(0, ROWS_PER_W)],
                    out_ref.at[pl.ds(lo, ROWS_PER_W)])


def _edge_min(table, dst_idx, src_idx):
    compact = pl.kernel(
        _compact_body,
        out_type=(
            jax.ShapeDtypeStruct((NW * MAXBLK * KB,), jnp.int32),
            jax.ShapeDtypeStruct((NW * MAXBLK * KB,), jnp.int32),
            jax.ShapeDtypeStruct((NW * L,), jnp.int32),
        ),
        mesh=plsc.VectorSubcoreMesh(core_axis_name="c", subcore_axis_name="s"),
        scratch_types=[
            pltpu.VMEM((KB,), jnp.int32),
            pltpu.VMEM((KB,), jnp.int32),
            pltpu.VMEM((CHUNK,), jnp.int32),
            pltpu.VMEM((CHUNK,), jnp.int32),
            pltpu.VMEM((L,), jnp.int32),
        ],
        compiler_params=pltpu.CompilerParams(needs_layout_passes=False),
        name="edge_compact",
    )
    dlist, slist, counts = compact(dst_idx, src_idx)

    rmw = pl.kernel(
        _rmw_min_body,
        out_type=jax.ShapeDtypeStruct((N_PAD, H), jnp.float32),
        mesh=plsc.VectorSubcoreMesh(core_axis_name="c", subcore_axis_name="s"),
        scratch_types=[
            pltpu.VMEM((ACC_ROWS, H), jnp.float32),
            pltpu.VMEM((KB, H), jnp.float32),
            pltpu.VMEM((KB,), jnp.int32),
            pltpu.VMEM((8, 128), jnp.int32),
            pltpu.VMEM((L,), jnp.int32),
            pltpu.SemaphoreType.DMA,
        ],
        compiler_params=pltpu.CompilerParams(needs_layout_passes=False),
        name="edge_rmw_min",
    )
    return rmw(table, dlist, slist, counts)


def _edge_sum_body(table_ref, dst_hbm, src_hbm, out_ref,
                   acc_sh, rows, dbuf, sbuf, zbuf, sem):
    cid = lax.axis_index("c")
    sid = lax.axis_index("s")
    wid = sid * NC + cid

    zrow = jnp.zeros((L,), dtype=jnp.float32)
    def zinit(r, _):
        for j in range(H // L):
            zbuf[r, pl.ds(j * L, L)] = zrow
        return 0
    lax.fori_loop(0, ZROWS, zinit, 0)
    for q in range(STRIPE // ZROWS):
        pltpu.sync_copy(zbuf, acc_sh.at[pl.ds(sid * STRIPE + q * ZROWS,
                                              ZROWS)])
    @pl.when(sid == 0)
    def _():
        rem = NY_ACC - 16 * STRIPE
        pltpu.sync_copy(zbuf.at[pl.ds(0, rem)],
                        acc_sh.at[pl.ds(16 * STRIPE, rem)])
    plsc.subcore_barrier()

    def block_body(i, _):
        blk = i * NW + wid

        @pl.when(blk < N_BLOCKS)
        def _():
            r0 = blk * SUM_BLK_ROWS
            pltpu.sync_copy(src_hbm.at[pl.ds(r0, SUM_BLK_ROWS)], sbuf)
            pltpu.sync_copy(dst_hbm.at[pl.ds(r0, SUM_BLK_ROWS)], dbuf)
            for quarter in range(4):
                descs = []
                for g in range(2):
                    descs.append(pltpu.async_copy(
                        table_ref.at[sbuf.at[2 * quarter + g]],
                        rows.at[pl.ds(g * 128, 128)], sem))
                for d in descs:
                    d.wait()
                for g in range(2):
                    pltpu.sync_copy(rows.at[pl.ds(g * 128, 128)],
                                    acc_sh.at[dbuf.at[2 * quarter + g]],
                                    add=True)
        return 0

    lax.fori_loop(0, MAX_BLOCKS_PER_W, block_body, 0)
    plsc.subcore_barrier()

    pltpu.sync_copy(acc_sh.at[pl.ds(sid * STRIPE, STRIPE)],
                    out_ref.at[cid].at[pl.ds(sid * STRIPE, STRIPE)])
    @pl.when(sid == 0)
    def _():
        rem = NY_ACC - 16 * STRIPE
        pltpu.sync_copy(acc_sh.at[pl.ds(16 * STRIPE, rem)],
                        out_ref.at[cid].at[pl.ds(16 * STRIPE, rem)])


def _edge_sum(table, dst_idx2d, src_idx2d):
    f = pl.kernel(
        _edge_sum_body,
        out_type=jax.ShapeDtypeStruct((NC, NY_ACC, H), jnp.float32),
        mesh=plsc.VectorSubcoreMesh(core_axis_name="c", subcore_axis_name="s"),
        scratch_types=[
            pltpu.VMEM_SHARED((NY_ACC, H), jnp.float32),
            pltpu.VMEM((256, H), jnp.float32),
            pltpu.VMEM((SUM_BLK_ROWS, 128), jnp.int32),
            pltpu.VMEM((SUM_BLK_ROWS, 128), jnp.int32),
            pltpu.VMEM((ZROWS, 128), jnp.float32),
            pltpu.SemaphoreType.DMA,
        ],
        compiler_params=pltpu.CompilerParams(needs_layout_passes=False),
        name="edge_sum",
    )
    return f(table, dst_idx2d, src_idx2d)


def kernel(h_x, h_x_degree, edge_index, x_mask, y_mask, edge_mask,
           batch_index_x, batch_index_y, batch_size,
           W1, b1, W2, b2, W3, b3, W4, b4):
    num_y = y_mask.shape[0]
    num_x = x_mask.shape[0]
    ei0 = jnp.where(edge_mask, edge_index[0], num_y)
    ei1 = jnp.where(edge_mask, edge_index[1], num_x)

    x_degree = _mlp_pallas(h_x, [h_x_degree], W1, b1, W2, b2)

    next_y_pad = _edge_min(x_degree, ei0, edge_index[1])
    next_y = next_y_pad[:num_y]

    pad_n = E_IDX_ROWS_PAD * 128 - E
    dst2d = jnp.concatenate(
        [ei1, jnp.full((pad_n,), NY_ACC - 1, jnp.int32)]).reshape(-1, 128)
    src2d = jnp.concatenate(
        [edge_index[0], jnp.zeros((pad_n,), jnp.int32)]).reshape(-1, 128)
    partials = _edge_sum(next_y, dst2d, src2d)

    next_x = _mlp_pallas(
        h_x, [partials[0, :num_x], partials[1, :num_x]], W3, b3, W4, b4)
    return (next_x, next_y)


# speedup vs baseline: 1.6980x; 1.6980x over previous
import functools

import jax
import jax.numpy as jnp
from jax import lax
from jax.experimental import pallas as pl
from jax.experimental.pallas import tpu as pltpu
from jax.experimental.pallas import tpu_sc as plsc

H = 128
NX = 10000
E = 320000
ROW_BLOCK = 2000

NC = 2
NS = 16
NW = NC * NS
L = 16

ROWS_PER_W = 320
N_PAD = NW * ROWS_PER_W
TRASH = ROWS_PER_W
ACC_ROWS = ROWS_PER_W + 8
KBUF = 512
UNROLL = 2
FLUSH_AT = KBUF - UNROLL * L
CHUNK = 4000
N_CHUNKS = E // CHUNK

SUM_BLK_ROWS = 8
SUM_BLK = SUM_BLK_ROWS * 128
E_IDX_ROWS = -(-E // 128)
N_BLOCKS = -(-E_IDX_ROWS // SUM_BLK_ROWS)
E_IDX_ROWS_PAD = N_BLOCKS * SUM_BLK_ROWS
MAX_BLOCKS_PER_W = -(-N_BLOCKS // NW)
NY_ACC = 10016
STRIPE = 624
ZROWS = 78


def _mlp_block_kernel(a_ref, b_ref, w1t_ref, b1_ref, w2t_ref, b2_ref, o_ref,
                      *, two_b):
    a = a_ref[...]
    b = b_ref[0][...] + b_ref[1][...] if two_b else b_ref[0][...]
    h = jnp.dot(a, w1t_ref[:H, :], preferred_element_type=jnp.float32)
    h += jnp.dot(b, w1t_ref[H:, :], preferred_element_type=jnp.float32)
    h = jnp.maximum(h + b1_ref[...], 0.0)
    o = jnp.dot(h, w2t_ref[...], preferred_element_type=jnp.float32)
    o_ref[...] = jnp.maximum(o + b2_ref[...], 0.0).astype(o_ref.dtype)


def _mlp_pallas(a, bs, W1, b1, W2, b2, out_dtype=jnp.float32):
    n = a.shape[0]
    grid = n // ROW_BLOCK
    two_b = len(bs) == 2
    bspec = pl.BlockSpec((ROW_BLOCK, H), lambda i: (i, 0))
    return pl.pallas_call(
        functools.partial(_mlp_block_kernel, two_b=two_b),
        grid=(grid,),
        in_specs=[
            bspec,
            [bspec] * len(bs),
            pl.BlockSpec((2 * H, 2 * H), lambda i: (0, 0)),
            pl.BlockSpec((1, 2 * H), lambda i: (0, 0)),
            pl.BlockSpec((2 * H, H), lambda i: (0, 0)),
            pl.BlockSpec((1, H), lambda i: (0, 0)),
        ],
        out_specs=bspec,
        out_shape=jax.ShapeDtypeStruct((n, H), out_dtype),
    )(a, list(bs), W1.T, b1[None, :], W2.T, b2[None, :])


def _edge_min_body(table_ref, dst_hbm, src_hbm, out_ref,
                   acc, rows, dstbuf, srcbuf, idx2d, dbuf, sbuf, sem):
    wid = lax.axis_index("s") * NC + lax.axis_index("c")
    lo = wid * ROWS_PER_W
    inf_vec = jnp.full((L,), jnp.inf, dtype=jnp.float32)
    trash_vec = jnp.full((L,), TRASH, dtype=jnp.int32)
    zero_vec = jnp.zeros((L,), dtype=jnp.int32)

    def init_acc(r, _):
        for j in range(H // L):
            acc[r, pl.ds(j * L, L)] = inf_vec
        return 0
    lax.fori_loop(0, ACC_ROWS, init_acc, 0)

    def init_bufs(i, _):
        dstbuf[pl.ds(i * L, L)] = trash_vec
        srcbuf[pl.ds(i * L, L)] = zero_vec
        return 0
    lax.fori_loop(0, KBUF // L, init_bufs, 0)

    def flush():
        for g in range(KBUF // 128):
            for j in range(128 // L):
                idx2d[g, pl.ds(j * L, L)] = srcbuf[pl.ds(g * 128 + j * L, L)]
        descs = []
        for g in range(KBUF // 128):
            descs.append(pltpu.async_copy(
                table_ref.at[idx2d.at[g]],
                rows.at[pl.ds(g * 128, 128)], sem))
        for d in descs:
            d.wait()

        def rmw_group(g, _):
            dvec = dstbuf[pl.ds(g * L, L)]
            for i in range(L):
                d = dvec[i]
                k = g * L + i
                for j in range(H // L):
                    sl = pl.ds(j * L, L)
                    acc[d, sl] = jnp.minimum(acc[d, sl], rows[k, sl])
            return 0
        lax.fori_loop(0, KBUF // L, rmw_group, 0)
        lax.fori_loop(0, KBUF // L, init_bufs, 0)

    def chunk_body(c, off):
        pltpu.sync_copy(dst_hbm.at[pl.ds(c * CHUNK, CHUNK)], dbuf)
        pltpu.sync_copy(src_hbm.at[pl.ds(c * CHUNK, CHUNK)], sbuf)

        def step(t, off):
            def do_flush():
                flush()
                return jnp.int32(0)
            off = lax.cond(off >= FLUSH_AT, do_flush, lambda: off)
            for u in range(UNROLL):
                base = t * (UNROLL * L) + u * L
                dv = dbuf[pl.ds(base, L)]
                sv = sbuf[pl.ds(base, L)]
                m = (dv >= lo) & (dv < lo + ROWS_PER_W)
                cnt = plsc.all_reduce_population_count(m)[0]
                plsc.store_compressed(dstbuf.at[pl.ds(off, L)], dv - lo,
                                      mask=m)
                plsc.store_compressed(srcbuf.at[pl.ds(off, L)], sv, mask=m)
                off = off + cnt
            return off

        return lax.fori_loop(0, CHUNK // (UNROLL * L), step, off)

    lax.fori_loop(0, N_CHUNKS, chunk_body, jnp.int32(0))
    flush()

    inf = jnp.float32(jnp.inf)
    def finalize(r, _):
        for j in range(H // L):
            sl = pl.ds(j * L, L)
            v = acc[r, sl]
            acc[r, sl] = jnp.where(v < inf, v, 0.0)
        return 0
    lax.fori_loop(0, ROWS_PER_W, finalize, 0)

    pltpu.sync_copy(acc.at[pl.ds(0, ROWS_PER_W)],
                    out_ref.at[pl.ds(lo, ROWS_PER_W)])


def _edge_min(table, dst_idx, src_idx):
    f = pl.kernel(
        _edge_min_body,
        out_type=jax.ShapeDtypeStruct((N_PAD, H), jnp.float32),
        mesh=plsc.VectorSubcoreMesh(core_axis_name="c", subcore_axis_name="s"),
        scratch_types=[
            pltpu.VMEM((ACC_ROWS, H), jnp.float32),
            pltpu.VMEM((KBUF, H), jnp.float32),
            pltpu.VMEM((KBUF,), jnp.int32),
            pltpu.VMEM((KBUF,), jnp.int32),
            pltpu.VMEM((KBUF // 128, 128), jnp.int32),
            pltpu.VMEM((CHUNK,), jnp.int32),
            pltpu.VMEM((CHUNK,), jnp.int32),
            pltpu.SemaphoreType.DMA,
        ],
        compiler_params=pltpu.CompilerParams(needs_layout_passes=False),
        name="edge_min",
    )
    return f(table, dst_idx, src_idx)


def _edge_sum_body(table_ref, dst_hbm, src_hbm, out_ref,
                   acc_sh, rows, dbuf, sbuf, zbuf, sem):
    cid = lax.axis_index("c")
    sid = lax.axis_index("s")
    wid = sid * NC + cid

    zrow = jnp.zeros((L,), dtype=jnp.float32)
    def zinit(r, _):
        for j in range(H // L):
            zbuf[r, pl.ds(j * L, L)] = zrow
        return 0
    lax.fori_loop(0, ZROWS, zinit, 0)
    for q in range(STRIPE // ZROWS):
        pltpu.sync_copy(zbuf, acc_sh.at[pl.ds(sid * STRIPE + q * ZROWS,
                                              ZROWS)])
    @pl.when(sid == 0)
    def _():
        rem = NY_ACC - 16 * STRIPE
        pltpu.sync_copy(zbuf.at[pl.ds(0, rem)],
                        acc_sh.at[pl.ds(16 * STRIPE, rem)])
    plsc.subcore_barrier()

    def block_body(i, _):
        blk = i * NW + wid

        @pl.when(blk < N_BLOCKS)
        def _():
            r0 = blk * SUM_BLK_ROWS
            pltpu.sync_copy(src_hbm.at[pl.ds(r0, SUM_BLK_ROWS)], sbuf)
            pltpu.sync_copy(dst_hbm.at[pl.ds(r0, SUM_BLK_ROWS)], dbuf)
            for quarter in range(4):
                descs = []
                for g in range(2):
                    descs.append(pltpu.async_copy(
                        table_ref.at[sbuf.at[2 * quarter + g]],
                        rows.at[pl.ds(g * 128, 128)], sem))
                for d in descs:
                    d.wait()
                for g in range(2):
                    pltpu.sync_copy(rows.at[pl.ds(g * 128, 128)],
                                    acc_sh.at[dbuf.at[2 * quarter + g]],
                                    add=True)
        return 0

    lax.fori_loop(0, MAX_BLOCKS_PER_W, block_body, 0)
    plsc.subcore_barrier()

    pltpu.sync_copy(acc_sh.at[pl.ds(sid * STRIPE, STRIPE)],
                    out_ref.at[cid].at[pl.ds(sid * STRIPE, STRIPE)])
    @pl.when(sid == 0)
    def _():
        rem = NY_ACC - 16 * STRIPE
        pltpu.sync_copy(acc_sh.at[pl.ds(16 * STRIPE, rem)],
                        out_ref.at[cid].at[pl.ds(16 * STRIPE, rem)])


def _edge_sum(table, dst_idx2d, src_idx2d):
    f = pl.kernel(
        _edge_sum_body,
        out_type=jax.ShapeDtypeStruct((NC, NY_ACC, H), jnp.float32),
        mesh=plsc.VectorSubcoreMesh(core_axis_name="c", subcore_axis_name="s"),
        scratch_types=[
            pltpu.VMEM_SHARED((NY_ACC, H), jnp.float32),
            pltpu.VMEM((256, H), jnp.float32),
            pltpu.VMEM((SUM_BLK_ROWS, 128), jnp.int32),
            pltpu.VMEM((SUM_BLK_ROWS, 128), jnp.int32),
            pltpu.VMEM((ZROWS, 128), jnp.float32),
            pltpu.SemaphoreType.DMA,
        ],
        compiler_params=pltpu.CompilerParams(needs_layout_passes=False),
        name="edge_sum",
    )
    return f(table, dst_idx2d, src_idx2d)


def kernel(h_x, h_x_degree, edge_index, x_mask, y_mask, edge_mask,
           batch_index_x, batch_index_y, batch_size,
           W1, b1, W2, b2, W3, b3, W4, b4):
    num_y = y_mask.shape[0]
    num_x = x_mask.shape[0]
    ei0 = jnp.where(edge_mask, edge_index[0], num_y)
    ei1 = jnp.where(edge_mask, edge_index[1], num_x)

    x_degree = _mlp_pallas(h_x, [h_x_degree], W1, b1, W2, b2)

    next_y_pad = _edge_min(x_degree, ei0, edge_index[1])
    next_y = next_y_pad[:num_y]

    pad_n = E_IDX_ROWS_PAD * 128 - E
    dst2d = jnp.concatenate(
        [ei1, jnp.full((pad_n,), NY_ACC - 1, jnp.int32)]).reshape(-1, 128)
    src2d = jnp.concatenate(
        [edge_index[0], jnp.zeros((pad_n,), jnp.int32)]).reshape(-1, 128)
    partials = _edge_sum(next_y, dst2d, src2d)

    next_x = _mlp_pallas(
        h_x, [partials[0, :num_x], partials[1, :num_x]], W3, b3, W4, b4)
    return (next_x, next_y)
